# baseline (speedup 1.0000x reference)
import functools

import jax
import jax.numpy as jnp
from jax import lax
from jax.experimental import pallas as pl
from jax.experimental.pallas import tpu as pltpu
from jax.experimental.pallas import tpu_sc as plsc

NC = 2
NS = 16
L = 16
EPS2 = 1e-24


def kernel(q, t, indices):
    B = indices.shape[0]
    NW = NC * NS
    BPW = B // NW
    NG = BPW // L

    mesh = plsc.VectorSubcoreMesh(
        core_axis_name="c", subcore_axis_name="s", num_cores=NC, num_subcores=NS
    )

    @functools.partial(
        pl.kernel,
        out_type=jax.ShapeDtypeStruct((B * 16,), jnp.float32),
        mesh=mesh,
        scratch_types=[
            pltpu.VMEM((BPW,), jnp.int32),
            [pltpu.VMEM((BPW,), jnp.int32) for _ in range(7)],
            [pltpu.VMEM((BPW,), jnp.float32) for _ in range(7)],
            pltpu.VMEM((BPW * 16,), jnp.float32),
            pltpu.SemaphoreType.DMA,
        ],
        compiler_params=pltpu.CompilerParams(needs_layout_passes=False),
    )
    def pose_kernel(q_hbm, t_hbm, idx_hbm, out_hbm, idx_v, idxc, comp, out_v, sem):
        wid = lax.axis_index("s") * NC + lax.axis_index("c")
        base = wid * BPW
        pltpu.sync_copy(idx_hbm.at[pl.ds(base, BPW)], idx_v)

        for g in range(NG):
            sl = pl.ds(g * L, L)
            ii = idx_v[sl]
            i4 = ii * 4
            i3 = ii * 3
            idxc[0][sl] = i4
            idxc[1][sl] = i4 + 1
            idxc[2][sl] = i4 + 2
            idxc[3][sl] = i4 + 3
            idxc[4][sl] = i3
            idxc[5][sl] = i3 + 1
            idxc[6][sl] = i3 + 2

        copies = []
        for c in range(4):
            copies.append(pltpu.async_copy(q_hbm.at[idxc[c]], comp[c], sem))
        for c in range(3):
            copies.append(pltpu.async_copy(t_hbm.at[idxc[4 + c]], comp[4 + c], sem))
        for cp in copies:
            cp.wait()

        lane = lax.iota(jnp.int32, L)
        zeros = jnp.zeros((L,), jnp.float32)
        ones = jnp.ones((L,), jnp.float32)

        for g in range(NG):
            sl = pl.ds(g * L, L)
            x = comp[0][sl]
            y = comp[1][sl]
            z = comp[2][sl]
            w = comp[3][sl]
            tx = comp[4][sl]
            ty = comp[5][sl]
            tz = comp[6][sl]

            xx = x * x
            yy = y * y
            zz = z * z
            ww = w * w
            n2 = xx + yy + zz + ww
            s = 2.0 / jnp.maximum(n2, EPS2)
            xy = x * y
            xz = x * z
            yz = y * z
            xw = x * w
            yw = y * w
            zw = z * w

            r00 = 1.0 - s * (yy + zz)
            r01 = s * (xy - zw)
            r02 = s * (xz + yw)
            r10 = s * (xy + zw)
            r11 = 1.0 - s * (xx + zz)
            r12 = s * (yz - xw)
            r20 = s * (xz - yw)
            r21 = s * (yz + xw)
            r22 = 1.0 - s * (xx + yy)

            offs = (lane + g * L) * 16
            plsc.store_scatter(out_v, [offs], r00)
            plsc.store_scatter(out_v, [offs + 1], r01)
            plsc.store_scatter(out_v, [offs + 2], r02)
            plsc.store_scatter(out_v, [offs + 3], tx)
            plsc.store_scatter(out_v, [offs + 4], r10)
            plsc.store_scatter(out_v, [offs + 5], r11)
            plsc.store_scatter(out_v, [offs + 6], r12)
            plsc.store_scatter(out_v, [offs + 7], ty)
            plsc.store_scatter(out_v, [offs + 8], r20)
            plsc.store_scatter(out_v, [offs + 9], r21)
            plsc.store_scatter(out_v, [offs + 10], r22)
            plsc.store_scatter(out_v, [offs + 11], tz)
            plsc.store_scatter(out_v, [offs + 12], zeros)
            plsc.store_scatter(out_v, [offs + 13], zeros)
            plsc.store_scatter(out_v, [offs + 14], zeros)
            plsc.store_scatter(out_v, [offs + 15], ones)

        pltpu.sync_copy(out_v, out_hbm.at[pl.ds(base * 16, BPW * 16)])

    flat = pose_kernel(q.reshape(-1), t.reshape(-1), indices)
    return flat.reshape(B, 4, 4)


# speedup vs baseline: 33.4835x; 33.4835x over previous
import functools

import jax
import jax.numpy as jnp
from jax import lax
from jax.experimental import pallas as pl
from jax.experimental.pallas import tpu as pltpu
from jax.experimental.pallas import tpu_sc as plsc

NC = 2
NS = 16
L = 16
EPS2 = 1e-24


def kernel(q, t, indices):
    B = indices.shape[0]
    NW = NC * NS
    BPW = B // NW
    NBLK = B // 128
    LBLK = BPW // 128
    NG = BPW // L

    mesh = plsc.VectorSubcoreMesh(
        core_axis_name="c", subcore_axis_name="s", num_cores=NC, num_subcores=NS
    )

    @functools.partial(
        pl.kernel,
        out_type=jax.ShapeDtypeStruct((B * 16,), jnp.float32),
        mesh=mesh,
        scratch_types=[
            pltpu.VMEM((BPW,), jnp.int32),
            [pltpu.VMEM((BPW,), jnp.float32) for _ in range(7)],
            pltpu.VMEM((BPW * 16,), jnp.float32),
            pltpu.SemaphoreType.DMA,
        ],
        compiler_params=pltpu.CompilerParams(needs_layout_passes=False),
    )
    def pose_kernel(x_h, y_h, z_h, w_h, tx_h, ty_h, tz_h, idx_hbm, out_hbm,
                    idx_v, comp, out_v, sem):
        wid = lax.axis_index("s") * NC + lax.axis_index("c")
        base = wid * BPW
        pltpu.sync_copy(idx_hbm.at[pl.ds(base, BPW)], idx_v)

        planes = (x_h, y_h, z_h, w_h, tx_h, ty_h, tz_h)
        copies = [
            pltpu.async_copy(p.at[idx_v], comp[c], sem)
            for c, p in enumerate(planes)
        ]
        for cp in copies:
            cp.wait()

        zeros = jnp.zeros((L,), jnp.float32)
        ones = jnp.ones((L,), jnp.float32)

        for g in range(NG):
            sl = pl.ds(g * L, L)
            x = comp[0][sl]
            y = comp[1][sl]
            z = comp[2][sl]
            w = comp[3][sl]
            tx = comp[4][sl]
            ty = comp[5][sl]
            tz = comp[6][sl]

            xx = x * x
            yy = y * y
            zz = z * z
            ww = w * w
            n2 = xx + yy + zz + ww
            s = 2.0 / jnp.maximum(n2, EPS2)
            xy = x * y
            xz = x * z
            yz = y * z
            xw = x * w
            yw = y * w
            zw = z * w

            lb = g // 8
            io = (g % 8) * L
            o = lb * 512 + io

            def st(r1, r2, val, o=o):
                out_v[pl.ds(r1 * (LBLK * 512) + o + r2 * 128, L)] = val

            st(0, 0, 1.0 - s * (yy + zz))
            st(0, 1, s * (xy - zw))
            st(0, 2, s * (xz + yw))
            st(0, 3, tx)
            st(1, 0, s * (xy + zw))
            st(1, 1, 1.0 - s * (xx + zz))
            st(1, 2, s * (yz - xw))
            st(1, 3, ty)
            st(2, 0, s * (xz - yw))
            st(2, 1, s * (yz + xw))
            st(2, 2, 1.0 - s * (xx + yy))
            st(2, 3, tz)
            st(3, 0, zeros)
            st(3, 1, zeros)
            st(3, 2, zeros)
            st(3, 3, ones)

        for r1 in range(4):
            pltpu.sync_copy(
                out_v.at[pl.ds(r1 * (LBLK * 512), LBLK * 512)],
                out_hbm.at[pl.ds(r1 * (NBLK * 512) + wid * (LBLK * 512),
                                 LBLK * 512)],
            )

    flat = pose_kernel(
        q[:, 0], q[:, 1], q[:, 2], q[:, 3], t[:, 0], t[:, 1], t[:, 2], indices
    )
    return (
        flat.reshape(4, NBLK, 4, 128)
        .transpose(1, 3, 0, 2)
        .reshape(B, 4, 4)
    )


# speedup vs baseline: 73.7552x; 2.2027x over previous
import functools

import jax
import jax.numpy as jnp
from jax import lax
from jax.experimental import pallas as pl
from jax.experimental.pallas import tpu as pltpu
from jax.experimental.pallas import tpu_sc as plsc

NC = 2
NS = 16
L = 16
EPS2 = 1e-24


def kernel(q, t, indices):
    B = indices.shape[0]
    N = q.shape[0]
    NW = NC * NS
    BPW = B // NW
    NBLK = B // 128
    LBLK = BPW // 128
    NG = BPW // L

    NFB = N // 128
    TAIL = N - NFB * 128
    WB = NFB // NW
    XBLK = NFB - WB * NW
    NCHUNK = 4
    CB = WB // NCHUNK
    CCOLS = CB * 128

    mesh = plsc.VectorSubcoreMesh(
        core_axis_name="c", subcore_axis_name="s", num_cores=NC, num_subcores=NS
    )
    plane_ty = jax.ShapeDtypeStruct((N,), jnp.float32)

    @functools.partial(
        pl.kernel,
        out_type=[plane_ty] * 7,
        mesh=mesh,
        scratch_types=[
            pltpu.VMEM((4, CCOLS), jnp.float32),
            pltpu.VMEM((4, CCOLS), jnp.float32),
        ],
        compiler_params=pltpu.CompilerParams(needs_layout_passes=False),
    )
    def sweep_kernel(qt_h, tt_h, qx_t, qy_t, qz_t, qw_t, tx_t, ty_t, tz_t,
                     px, py, pz, pw, ptx, pty, ptz, bufq, buft):
        wid = lax.axis_index("s") * NC + lax.axis_index("c")
        qouts = (px, py, pz, pw)
        touts = (ptx, pty, ptz)
        for k in range(NCHUNK):
            c0 = (wid * WB + k * CB) * 128
            pltpu.sync_copy(qt_h.at[:, pl.ds(c0, CCOLS)], bufq)
            for c in range(4):
                pltpu.sync_copy(bufq.at[c], qouts[c].at[pl.ds(c0, CCOLS)])
            pltpu.sync_copy(tt_h.at[:, pl.ds(c0, CCOLS)], buft.at[pl.ds(0, 3)])
            for c in range(3):
                pltpu.sync_copy(buft.at[c], touts[c].at[pl.ds(c0, CCOLS)])

        @pl.when(wid == NW - 1)
        def _extra():
            c0 = NW * WB * 128
            n = XBLK * 128
            pltpu.sync_copy(qt_h.at[:, pl.ds(c0, n)], bufq.at[:, pl.ds(0, n)])
            for c in range(4):
                pltpu.sync_copy(bufq.at[c, pl.ds(0, n)],
                                qouts[c].at[pl.ds(c0, n)])
            pltpu.sync_copy(tt_h.at[:, pl.ds(c0, n)],
                            buft.at[pl.ds(0, 3), pl.ds(0, n)])
            for c in range(3):
                pltpu.sync_copy(buft.at[c, pl.ds(0, n)],
                                touts[c].at[pl.ds(c0, n)])

        @pl.when(wid == 0)
        def _tails():
            c0 = NFB * 128
            stage = bufq.at[0, pl.ds(0, TAIL)]
            for c, tail in enumerate((qx_t, qy_t, qz_t, qw_t)):
                pltpu.sync_copy(tail, stage)
                pltpu.sync_copy(stage, qouts[c].at[pl.ds(c0, TAIL)])
            for c, tail in enumerate((tx_t, ty_t, tz_t)):
                pltpu.sync_copy(tail, stage)
                pltpu.sync_copy(stage, touts[c].at[pl.ds(c0, TAIL)])

    @functools.partial(
        pl.kernel,
        out_type=jax.ShapeDtypeStruct((B * 16,), jnp.float32),
        mesh=mesh,
        scratch_types=[
            pltpu.VMEM((BPW,), jnp.int32),
            [pltpu.VMEM((BPW,), jnp.float32) for _ in range(7)],
            pltpu.VMEM((BPW * 16,), jnp.float32),
            pltpu.SemaphoreType.DMA,
        ],
        compiler_params=pltpu.CompilerParams(needs_layout_passes=False),
    )
    def pose_kernel(x_h, y_h, z_h, w_h, tx_h, ty_h, tz_h, idx_hbm, out_hbm,
                    idx_v, comp, out_v, sem):
        wid = lax.axis_index("s") * NC + lax.axis_index("c")
        base = wid * BPW
        pltpu.sync_copy(idx_hbm.at[pl.ds(base, BPW)], idx_v)

        planes = (x_h, y_h, z_h, w_h, tx_h, ty_h, tz_h)
        copies = [
            pltpu.async_copy(p.at[idx_v], comp[c], sem)
            for c, p in enumerate(planes)
        ]
        for cp in copies:
            cp.wait()

        zeros = jnp.zeros((L,), jnp.float32)
        ones = jnp.ones((L,), jnp.float32)

        for g in range(NG):
            sl = pl.ds(g * L, L)
            x = comp[0][sl]
            y = comp[1][sl]
            z = comp[2][sl]
            w = comp[3][sl]
            tx = comp[4][sl]
            ty = comp[5][sl]
            tz = comp[6][sl]

            xx = x * x
            yy = y * y
            zz = z * z
            ww = w * w
            n2 = xx + yy + zz + ww
            s = 2.0 / jnp.maximum(n2, EPS2)
            xy = x * y
            xz = x * z
            yz = y * z
            xw = x * w
            yw = y * w
            zw = z * w

            lb = g // 8
            io = (g % 8) * L
            o = lb * 512 + io

            def st(r1, r2, val, o=o):
                out_v[pl.ds(r1 * (LBLK * 512) + o + r2 * 128, L)] = val

            st(0, 0, 1.0 - s * (yy + zz))
            st(0, 1, s * (xy - zw))
            st(0, 2, s * (xz + yw))
            st(0, 3, tx)
            st(1, 0, s * (xy + zw))
            st(1, 1, 1.0 - s * (xx + zz))
            st(1, 2, s * (yz - xw))
            st(1, 3, ty)
            st(2, 0, s * (xz - yw))
            st(2, 1, s * (yz + xw))
            st(2, 2, 1.0 - s * (xx + yy))
            st(2, 3, tz)
            st(3, 0, zeros)
            st(3, 1, zeros)
            st(3, 2, zeros)
            st(3, 3, ones)

        for r1 in range(4):
            pltpu.sync_copy(
                out_v.at[pl.ds(r1 * (LBLK * 512), LBLK * 512)],
                out_hbm.at[pl.ds(r1 * (NBLK * 512) + wid * (LBLK * 512),
                                 LBLK * 512)],
            )

    qt = q.T
    tt = t.T
    tail = N - TAIL
    planes = sweep_kernel(
        qt, tt,
        q[tail:, 0], q[tail:, 1], q[tail:, 2], q[tail:, 3],
        t[tail:, 0], t[tail:, 1], t[tail:, 2],
    )
    flat = pose_kernel(*planes, indices)
    return (
        flat.reshape(4, NBLK, 4, 128)
        .transpose(1, 3, 0, 2)
        .reshape(B, 4, 4)
    )


# speedup vs baseline: 82.3437x; 1.1164x over previous
import functools

import jax
import jax.numpy as jnp
from jax import lax
from jax.experimental import pallas as pl
from jax.experimental.pallas import tpu as pltpu
from jax.experimental.pallas import tpu_sc as plsc

NC = 2
NS = 16
L = 16
EPS2 = 1e-24


def kernel(q, t, indices):
    B = indices.shape[0]
    N = q.shape[0]
    NW = NC * NS
    BPW = B // NW
    NBLK = B // 128
    LBLK = BPW // 128
    NG = BPW // L

    NFB = N // 128
    TAIL = N - NFB * 128
    WB = NFB // NW
    XBLK = NFB - WB * NW
    NCHUNK = 4
    CB = WB // NCHUNK
    CCOLS = CB * 128

    mesh = plsc.VectorSubcoreMesh(
        core_axis_name="c", subcore_axis_name="s", num_cores=NC, num_subcores=NS
    )
    plane_ty = jax.ShapeDtypeStruct((N,), jnp.float32)

    @functools.partial(
        pl.kernel,
        out_type=[plane_ty] * 7,
        mesh=mesh,
        scratch_types=[
            [pltpu.VMEM((4, CCOLS), jnp.float32) for _ in range(2)],
            [pltpu.VMEM((4, CCOLS), jnp.float32) for _ in range(2)],
            [pltpu.SemaphoreType.DMA for _ in range(2)],
            [pltpu.SemaphoreType.DMA for _ in range(2)],
        ],
        compiler_params=pltpu.CompilerParams(needs_layout_passes=False),
    )
    def sweep_kernel(qt_h, tt_h, qx_t, qy_t, qz_t, qw_t, tx_t, ty_t, tz_t,
                     px, py, pz, pw, ptx, pty, ptz, bufq, buft, sin, sout):
        wid = lax.axis_index("s") * NC + lax.axis_index("c")
        qouts = (px, py, pz, pw)
        touts = (ptx, pty, ptz)

        def col0(k):
            return (wid * WB + k * CB) * 128

        def start_in(k):
            b = k % 2
            return (
                pltpu.async_copy(qt_h.at[:, pl.ds(col0(k), CCOLS)],
                                 bufq[b], sin[b]),
                pltpu.async_copy(tt_h.at[:, pl.ds(col0(k), CCOLS)],
                                 buft[b].at[pl.ds(0, 3)], sin[b]),
            )

        def start_outs(k):
            b = k % 2
            cps = []
            for c in range(4):
                cps.append(pltpu.async_copy(
                    bufq[b].at[c], qouts[c].at[pl.ds(col0(k), CCOLS)], sout[b]))
            for c in range(3):
                cps.append(pltpu.async_copy(
                    buft[b].at[c], touts[c].at[pl.ds(col0(k), CCOLS)], sout[b]))
            return cps

        ins = {0: start_in(0)}
        outs = {}
        for k in range(NCHUNK):
            for cp in ins.pop(k):
                cp.wait()
            if k >= 1:
                for cp in outs.pop(k - 1):
                    cp.wait()
            if k < NCHUNK - 1:
                ins[k + 1] = start_in(k + 1)
            outs[k] = start_outs(k)
        for cp in outs.pop(NCHUNK - 1):
            cp.wait()

        @pl.when(wid == NW - 1)
        def _extra():
            c0 = NW * WB * 128
            n = XBLK * 128
            bufq0, buft0 = bufq[0], buft[0]
            pltpu.sync_copy(qt_h.at[:, pl.ds(c0, n)], bufq0.at[:, pl.ds(0, n)])
            for c in range(4):
                pltpu.sync_copy(bufq0.at[c, pl.ds(0, n)],
                                qouts[c].at[pl.ds(c0, n)])
            pltpu.sync_copy(tt_h.at[:, pl.ds(c0, n)],
                            buft0.at[pl.ds(0, 3), pl.ds(0, n)])
            for c in range(3):
                pltpu.sync_copy(buft0.at[c, pl.ds(0, n)],
                                touts[c].at[pl.ds(c0, n)])

        @pl.when(wid == 0)
        def _tails():
            c0 = NFB * 128
            stage = bufq[0].at[0, pl.ds(0, TAIL)]
            for c, tail in enumerate((qx_t, qy_t, qz_t, qw_t)):
                pltpu.sync_copy(tail, stage)
                pltpu.sync_copy(stage, qouts[c].at[pl.ds(c0, TAIL)])
            for c, tail in enumerate((tx_t, ty_t, tz_t)):
                pltpu.sync_copy(tail, stage)
                pltpu.sync_copy(stage, touts[c].at[pl.ds(c0, TAIL)])

    @functools.partial(
        pl.kernel,
        out_type=jax.ShapeDtypeStruct((B * 16,), jnp.float32),
        mesh=mesh,
        scratch_types=[
            pltpu.VMEM((BPW,), jnp.int32),
            [pltpu.VMEM((BPW,), jnp.float32) for _ in range(7)],
            pltpu.VMEM((BPW * 16,), jnp.float32),
            pltpu.SemaphoreType.DMA,
        ],
        compiler_params=pltpu.CompilerParams(needs_layout_passes=False),
    )
    def pose_kernel(x_h, y_h, z_h, w_h, tx_h, ty_h, tz_h, idx_hbm, out_hbm,
                    idx_v, comp, out_v, sem):
        wid = lax.axis_index("s") * NC + lax.axis_index("c")
        base = wid * BPW
        pltpu.sync_copy(idx_hbm.at[pl.ds(base, BPW)], idx_v)

        planes = (x_h, y_h, z_h, w_h, tx_h, ty_h, tz_h)
        copies = [
            pltpu.async_copy(p.at[idx_v], comp[c], sem)
            for c, p in enumerate(planes)
        ]
        for cp in copies:
            cp.wait()

        zeros = jnp.zeros((L,), jnp.float32)
        ones = jnp.ones((L,), jnp.float32)

        for g in range(NG):
            sl = pl.ds(g * L, L)
            x = comp[0][sl]
            y = comp[1][sl]
            z = comp[2][sl]
            w = comp[3][sl]
            tx = comp[4][sl]
            ty = comp[5][sl]
            tz = comp[6][sl]

            xx = x * x
            yy = y * y
            zz = z * z
            ww = w * w
            n2 = xx + yy + zz + ww
            s = 2.0 / jnp.maximum(n2, EPS2)
            xy = x * y
            xz = x * z
            yz = y * z
            xw = x * w
            yw = y * w
            zw = z * w

            lb = g // 8
            io = (g % 8) * L
            o = lb * 512 + io

            def st(r1, r2, val, o=o):
                out_v[pl.ds(r1 * (LBLK * 512) + o + r2 * 128, L)] = val

            st(0, 0, 1.0 - s * (yy + zz))
            st(0, 1, s * (xy - zw))
            st(0, 2, s * (xz + yw))
            st(0, 3, tx)
            st(1, 0, s * (xy + zw))
            st(1, 1, 1.0 - s * (xx + zz))
            st(1, 2, s * (yz - xw))
            st(1, 3, ty)
            st(2, 0, s * (xz - yw))
            st(2, 1, s * (yz + xw))
            st(2, 2, 1.0 - s * (xx + yy))
            st(2, 3, tz)
            st(3, 0, zeros)
            st(3, 1, zeros)
            st(3, 2, zeros)
            st(3, 3, ones)

        for r1 in range(4):
            pltpu.sync_copy(
                out_v.at[pl.ds(r1 * (LBLK * 512), LBLK * 512)],
                out_hbm.at[pl.ds(r1 * (NBLK * 512) + wid * (LBLK * 512),
                                 LBLK * 512)],
            )

    qt = q.T
    tt = t.T
    tail = N - TAIL
    planes = sweep_kernel(
        qt, tt,
        q[tail:, 0], q[tail:, 1], q[tail:, 2], q[tail:, 3],
        t[tail:, 0], t[tail:, 1], t[tail:, 2],
    )
    flat = pose_kernel(*planes, indices)
    return (
        flat.reshape(4, NBLK, 4, 128)
        .transpose(1, 3, 0, 2)
        .reshape(B, 4, 4)
    )


# speedup vs baseline: 83.8409x; 1.0182x over previous
import functools

import jax
import jax.numpy as jnp
from jax import lax
from jax.experimental import pallas as pl
from jax.experimental.pallas import tpu as pltpu
from jax.experimental.pallas import tpu_sc as plsc

NC = 2
NS = 16
L = 16
EPS2 = 1e-24


def kernel(q, t, indices):
    B = indices.shape[0]
    N = q.shape[0]
    NW = NC * NS
    BPW = B // NW
    NBLK = B // 128
    LBLK = BPW // 128
    NG = BPW // L

    NFB = N // 128
    TAIL = N - NFB * 128
    WB = NFB // NW
    XBLK = NFB - WB * NW
    NCHUNK = 4
    CB = WB // NCHUNK
    CCOLS = CB * 128

    mesh = plsc.VectorSubcoreMesh(
        core_axis_name="c", subcore_axis_name="s", num_cores=NC, num_subcores=NS
    )
    plane_ty = jax.ShapeDtypeStruct((N,), jnp.float32)

    @functools.partial(
        pl.kernel,
        out_type=[plane_ty] * 7,
        mesh=mesh,
        scratch_types=[
            [pltpu.VMEM((4, CCOLS), jnp.float32) for _ in range(2)],
            [pltpu.VMEM((4, CCOLS), jnp.float32) for _ in range(2)],
            [pltpu.SemaphoreType.DMA for _ in range(2)],
            [pltpu.SemaphoreType.DMA for _ in range(2)],
        ],
        compiler_params=pltpu.CompilerParams(needs_layout_passes=False),
    )
    def sweep_kernel(qt_h, tt_h, qx_t, qy_t, qz_t, qw_t, tx_t, ty_t, tz_t,
                     px, py, pz, pw, ptx, pty, ptz, bufq, buft, sin, sout):
        wid = lax.axis_index("s") * NC + lax.axis_index("c")
        qouts = (px, py, pz, pw)
        touts = (ptx, pty, ptz)

        def col0(k):
            return (wid * WB + k * CB) * 128

        def start_in(k):
            b = k % 2
            return (
                pltpu.async_copy(qt_h.at[:, pl.ds(col0(k), CCOLS)],
                                 bufq[b], sin[b]),
                pltpu.async_copy(tt_h.at[:, pl.ds(col0(k), CCOLS)],
                                 buft[b].at[pl.ds(0, 3)], sin[b]),
            )

        def start_outs(k):
            b = k % 2
            cps = []
            for c in range(4):
                cps.append(pltpu.async_copy(
                    bufq[b].at[c], qouts[c].at[pl.ds(col0(k), CCOLS)], sout[b]))
            for c in range(3):
                cps.append(pltpu.async_copy(
                    buft[b].at[c], touts[c].at[pl.ds(col0(k), CCOLS)], sout[b]))
            return cps

        ins = {0: start_in(0)}
        outs = {}
        for k in range(NCHUNK):
            for cp in ins.pop(k):
                cp.wait()
            if k >= 1:
                for cp in outs.pop(k - 1):
                    cp.wait()
            if k < NCHUNK - 1:
                ins[k + 1] = start_in(k + 1)
            outs[k] = start_outs(k)
        for cp in outs.pop(NCHUNK - 1):
            cp.wait()

        @pl.when(wid == NW - 1)
        def _extra():
            c0 = NW * WB * 128
            n = XBLK * 128
            bufq0, buft0 = bufq[0], buft[0]
            pltpu.sync_copy(qt_h.at[:, pl.ds(c0, n)], bufq0.at[:, pl.ds(0, n)])
            for c in range(4):
                pltpu.sync_copy(bufq0.at[c, pl.ds(0, n)],
                                qouts[c].at[pl.ds(c0, n)])
            pltpu.sync_copy(tt_h.at[:, pl.ds(c0, n)],
                            buft0.at[pl.ds(0, 3), pl.ds(0, n)])
            for c in range(3):
                pltpu.sync_copy(buft0.at[c, pl.ds(0, n)],
                                touts[c].at[pl.ds(c0, n)])

        @pl.when(wid == 0)
        def _tails():
            c0 = NFB * 128
            stage = bufq[0].at[0, pl.ds(0, TAIL)]
            for c, tail in enumerate((qx_t, qy_t, qz_t, qw_t)):
                pltpu.sync_copy(tail, stage)
                pltpu.sync_copy(stage, qouts[c].at[pl.ds(c0, TAIL)])
            for c, tail in enumerate((tx_t, ty_t, tz_t)):
                pltpu.sync_copy(tail, stage)
                pltpu.sync_copy(stage, touts[c].at[pl.ds(c0, TAIL)])

    @functools.partial(
        pl.kernel,
        out_type=jax.ShapeDtypeStruct((B * 16,), jnp.float32),
        mesh=mesh,
        scratch_types=[
            pltpu.VMEM((BPW,), jnp.int32),
            [pltpu.VMEM((BPW,), jnp.float32) for _ in range(7)],
            pltpu.VMEM((BPW * 16,), jnp.float32),
            [pltpu.SemaphoreType.DMA for _ in range(2)],
        ],
        compiler_params=pltpu.CompilerParams(needs_layout_passes=False),
    )
    def pose_kernel(x_h, y_h, z_h, w_h, tx_h, ty_h, tz_h, idx_hbm, out_hbm,
                    idx_v, comp, out_v, sem):
        wid = lax.axis_index("s") * NC + lax.axis_index("c")
        base = wid * BPW
        H = BPW // 2
        pltpu.sync_copy(idx_hbm.at[pl.ds(base, BPW)], idx_v)

        planes = (x_h, y_h, z_h, w_h, tx_h, ty_h, tz_h)

        def start_gathers(h):
            sl = pl.ds(h * H, H)
            return [
                pltpu.async_copy(p.at[idx_v.at[sl]], comp[c].at[sl], sem[h])
                for c, p in enumerate(planes)
            ]

        g0 = start_gathers(0)
        g1 = start_gathers(1)
        for cp in g0:
            cp.wait()

        zeros = jnp.zeros((L,), jnp.float32)
        ones = jnp.ones((L,), jnp.float32)

        def compute_group(g):
            sl = pl.ds(g * L, L)
            x = comp[0][sl]
            y = comp[1][sl]
            z = comp[2][sl]
            w = comp[3][sl]
            tx = comp[4][sl]
            ty = comp[5][sl]
            tz = comp[6][sl]

            xx = x * x
            yy = y * y
            zz = z * z
            ww = w * w
            n2 = xx + yy + zz + ww
            s = 2.0 / jnp.maximum(n2, EPS2)
            xy = x * y
            xz = x * z
            yz = y * z
            xw = x * w
            yw = y * w
            zw = z * w

            lb = g // 8
            io = (g % 8) * L
            o = lb * 512 + io

            def st(r1, r2, val, o=o):
                out_v[pl.ds(r1 * (LBLK * 512) + o + r2 * 128, L)] = val

            st(0, 0, 1.0 - s * (yy + zz))
            st(0, 1, s * (xy - zw))
            st(0, 2, s * (xz + yw))
            st(0, 3, tx)
            st(1, 0, s * (xy + zw))
            st(1, 1, 1.0 - s * (xx + zz))
            st(1, 2, s * (yz - xw))
            st(1, 3, ty)
            st(2, 0, s * (xz - yw))
            st(2, 1, s * (yz + xw))
            st(2, 2, 1.0 - s * (xx + yy))
            st(2, 3, tz)
            st(3, 0, zeros)
            st(3, 1, zeros)
            st(3, 2, zeros)
            st(3, 3, ones)

        for g in range(NG // 2):
            compute_group(g)
        for cp in g1:
            cp.wait()
        for g in range(NG // 2, NG):
            compute_group(g)

        out_cps = []
        for r1 in range(4):
            out_cps.append(pltpu.async_copy(
                out_v.at[pl.ds(r1 * (LBLK * 512), LBLK * 512)],
                out_hbm.at[pl.ds(r1 * (NBLK * 512) + wid * (LBLK * 512),
                                 LBLK * 512)],
                sem[0],
            ))
        for cp in out_cps:
            cp.wait()

    qt = q.T
    tt = t.T
    tail = N - TAIL
    planes = sweep_kernel(
        qt, tt,
        q[tail:, 0], q[tail:, 1], q[tail:, 2], q[tail:, 3],
        t[tail:, 0], t[tail:, 1], t[tail:, 2],
    )
    flat = pose_kernel(*planes, indices)
    return (
        flat.reshape(4, NBLK, 4, 128)
        .transpose(1, 3, 0, 2)
        .reshape(B, 4, 4)
    )
